# baseline (speedup 1.0000x reference)
import functools

import jax
import jax.numpy as jnp
from jax.experimental import pallas as pl
from jax.experimental.pallas import tpu as pltpu

K = 1024
D = 64
BETA = 0.25
NB = 16
T = 32 * 32
TOK = NB * T
_INV_LN2 = 1.4426950408889634


def _body(x_ref, e_ref, zq_ref, l0_ref, l1_ref, l2_ref, l3_ref,
          counts_ref, loss_ref):
    step = pl.program_id(0)
    x = jnp.concatenate([x_ref[0].reshape(D, T), x_ref[1].reshape(D, T)],
                        axis=1)
    e = e_ref[...]

    a = jnp.sum(x * x, axis=0, keepdims=True)
    b = jnp.sum(e * e, axis=1)[:, None]
    c2 = jax.lax.dot_general(-2.0 * e, x, (((1,), (0,)), ((), ())),
                             precision=jax.lax.Precision.DEFAULT,
                             preferred_element_type=jnp.float32)
    dist = (a + b) + c2

    m = jnp.min(dist, axis=0, keepdims=True)
    ii = jax.lax.broadcasted_iota(jnp.int32, dist.shape, 0)
    idx = jnp.min(jnp.where(dist == m, ii, jnp.int32(K)), axis=0)

    onehot = (ii == idx[None, :]).astype(jnp.float32)
    zq = jax.lax.dot_general(e, onehot, (((0,), (0,)), ((), ())),
                             precision=jax.lax.Precision.DEFAULT,
                             preferred_element_type=jnp.float32)

    zq_st = x + (zq - x)
    zq_ref[0] = zq_st[:, :T]
    zq_ref[1] = zq_st[:, T:]

    @pl.when(step == 0)
    def _init():
        counts_ref[...] = jnp.zeros_like(counts_ref)
        loss_ref[0] = 0.0

    counts_ref[0, :] += jnp.sum(onehot, axis=1)
    loss_ref[0] += jnp.sum((zq - x) ** 2)

    @pl.when(step == pl.num_programs(0) - 1)
    def _finalize():
        counts = counts_ref[0, :]
        prob = counts / jnp.sum(counts)
        log_prob = jnp.log(prob + 1e-10) * jnp.float32(_INV_LN2)
        entropy_bits = -jnp.sum(prob * log_prob)
        words = jnp.exp(entropy_bits * jnp.float32(1.0 / _INV_LN2))
        e_loss = loss_ref[0] * jnp.float32(1.0 / (TOK * D))
        l0_ref[0, 0] = e_loss + BETA * e_loss
        l1_ref[0, 0] = e_loss
        l2_ref[0, 0] = e_loss
        l3_ref[0, 0] = words


@functools.partial(jax.jit, static_argnames=())
def kernel(inputs, E_weight):
    smem_out = jax.ShapeDtypeStruct((1, 1), jnp.float32)
    smem_spec = pl.BlockSpec((1, 1), lambda i: (0, 0), memory_space=pltpu.SMEM)
    zq3, l0, l1, l2, l3 = pl.pallas_call(
        _body,
        grid=(NB // 2,),
        in_specs=[pl.BlockSpec((2, D, 32, 32), lambda i: (i, 0, 0, 0)),
                  pl.BlockSpec((K, D), lambda i: (0, 0))],
        out_specs=[pl.BlockSpec((2, D, T), lambda i: (i, 0, 0)),
                   smem_spec, smem_spec, smem_spec, smem_spec],
        out_shape=[jax.ShapeDtypeStruct((NB, D, T), jnp.float32),
                   smem_out, smem_out, smem_out, smem_out],
        scratch_shapes=[pltpu.VMEM((1, K), jnp.float32),
                        pltpu.SMEM((1,), jnp.float32)],
    )(inputs, E_weight)

    Zq = zq3.reshape(NB, D, 32, 32)
    return (l0.reshape(()), Zq, l1.reshape(()), l2.reshape(()),
            l3.reshape(()))


# speedup vs baseline: 1.2909x; 1.2909x over previous
import functools

import jax
import jax.numpy as jnp
from jax.experimental import pallas as pl
from jax.experimental.pallas import tpu as pltpu

K = 1024
D = 64
BETA = 0.25
NB = 16
T = 32 * 32
TOK = NB * T
_INV_LN2 = 1.4426950408889634


def _body(x_ref, e_ref, zq_ref, l0_ref, l1_ref, l2_ref, l3_ref,
          counts_ref, loss_ref):
    step = pl.program_id(0)
    nimg = x_ref.shape[0]
    x = jnp.concatenate([x_ref[i] for i in range(nimg)], axis=1)
    e = e_ref[...]

    a = jnp.sum(x * x, axis=0, keepdims=True)
    b = jnp.sum(e * e, axis=1)[:, None]
    c2 = jax.lax.dot_general(-2.0 * e, x, (((1,), (0,)), ((), ())),
                             precision=jax.lax.Precision.DEFAULT,
                             preferred_element_type=jnp.float32)
    dist = (a + b) + c2

    m = jnp.min(dist, axis=0, keepdims=True)
    ii = jax.lax.broadcasted_iota(jnp.int32, dist.shape, 0)
    idx = jnp.min(jnp.where(dist == m, ii, jnp.int32(K)), axis=0)

    onehot = (ii == idx[None, :]).astype(jnp.float32)
    zq = jax.lax.dot_general(e, onehot, (((0,), (0,)), ((), ())),
                             precision=jax.lax.Precision.DEFAULT,
                             preferred_element_type=jnp.float32)

    zq_st = x + (zq - x)
    for i in range(nimg):
        zq_ref[i] = zq_st[:, i * T:(i + 1) * T]

    @pl.when(step == 0)
    def _init():
        counts_ref[...] = jnp.zeros_like(counts_ref)
        loss_ref[0] = 0.0

    counts_ref[0, :] += jnp.sum(onehot, axis=1)
    loss_ref[0] += jnp.sum((zq - x) ** 2)

    @pl.when(step == pl.num_programs(0) - 1)
    def _finalize():
        counts = counts_ref[0, :]
        prob = counts / jnp.sum(counts)
        log_prob = jnp.log(prob + 1e-10) * jnp.float32(_INV_LN2)
        entropy_bits = -jnp.sum(prob * log_prob)
        words = jnp.exp(entropy_bits * jnp.float32(1.0 / _INV_LN2))
        e_loss = loss_ref[0] * jnp.float32(1.0 / (TOK * D))
        l0_ref[0, 0] = e_loss + BETA * e_loss
        l1_ref[0, 0] = e_loss
        l2_ref[0, 0] = e_loss
        l3_ref[0, 0] = words


@functools.partial(jax.jit, static_argnames=())
def kernel(inputs, E_weight):
    x3 = inputs.reshape(NB, D, T)

    smem_out = jax.ShapeDtypeStruct((1, 1), jnp.float32)
    smem_spec = pl.BlockSpec((1, 1), lambda i: (0, 0), memory_space=pltpu.SMEM)
    zq3, l0, l1, l2, l3 = pl.pallas_call(
        _body,
        grid=(NB // 4,),
        in_specs=[pl.BlockSpec((4, D, T), lambda i: (i, 0, 0)),
                  pl.BlockSpec((K, D), lambda i: (0, 0))],
        out_specs=[pl.BlockSpec((4, D, T), lambda i: (i, 0, 0)),
                   smem_spec, smem_spec, smem_spec, smem_spec],
        out_shape=[jax.ShapeDtypeStruct((NB, D, T), jnp.float32),
                   smem_out, smem_out, smem_out, smem_out],
        scratch_shapes=[pltpu.VMEM((1, K), jnp.float32),
                        pltpu.SMEM((1,), jnp.float32)],
    )(x3, E_weight)

    Zq = zq3.reshape(NB, D, 32, 32)
    return (l0.reshape(()), Zq, l1.reshape(()), l2.reshape(()),
            l3.reshape(()))


# speedup vs baseline: 1.3160x; 1.0195x over previous
import functools

import jax
import jax.numpy as jnp
from jax.experimental import pallas as pl
from jax.experimental.pallas import tpu as pltpu

K = 1024
D = 64
BETA = 0.25
NB = 16
T = 32 * 32
TOK = NB * T
_INV_LN2 = 1.4426950408889634


def _body(x_ref, e_ref, zq_ref, l0_ref, l1_ref, l2_ref, l3_ref,
          counts_ref, loss_ref):
    step = pl.program_id(0)
    nimg = x_ref.shape[0]
    x = jnp.concatenate([x_ref[i] for i in range(nimg)], axis=1)
    e = e_ref[...]

    a = jnp.sum(x * x, axis=0, keepdims=True)
    b = jnp.sum(e * e, axis=1)[:, None]
    c2 = jax.lax.dot_general(-2.0 * e, x, (((1,), (0,)), ((), ())),
                             precision=jax.lax.Precision.DEFAULT,
                             preferred_element_type=jnp.float32)
    dist = (a + b) + c2

    m = jnp.min(dist, axis=0, keepdims=True)
    ii = jax.lax.broadcasted_iota(jnp.int32, dist.shape, 0)
    idx = jnp.min(jnp.where(dist == m, ii, jnp.int32(K)), axis=0)

    onehot = (ii == idx[None, :]).astype(jnp.float32)
    zq = jax.lax.dot_general(e, onehot, (((0,), (0,)), ((), ())),
                             precision=jax.lax.Precision.DEFAULT,
                             preferred_element_type=jnp.float32)

    zq_st = x + (zq - x)
    for i in range(nimg):
        zq_ref[i] = zq_st[:, i * T:(i + 1) * T]

    @pl.when(step == 0)
    def _init():
        counts_ref[...] = jnp.zeros_like(counts_ref)
        loss_ref[0] = 0.0

    counts_ref[0, :] += jnp.sum(onehot, axis=1)
    loss_ref[0] += jnp.sum((zq - x) ** 2)

    @pl.when(step == pl.num_programs(0) - 1)
    def _finalize():
        counts = counts_ref[0, :]
        prob = counts / jnp.sum(counts)
        log_prob = jnp.log(prob + 1e-10) * jnp.float32(_INV_LN2)
        entropy_bits = -jnp.sum(prob * log_prob)
        words = jnp.exp(entropy_bits * jnp.float32(1.0 / _INV_LN2))
        e_loss = loss_ref[0] * jnp.float32(1.0 / (TOK * D))
        l0_ref[0, 0] = e_loss + BETA * e_loss
        l1_ref[0, 0] = e_loss
        l2_ref[0, 0] = e_loss
        l3_ref[0, 0] = words


@functools.partial(jax.jit, static_argnames=())
def kernel(inputs, E_weight):
    x3 = inputs.reshape(NB, D, T)

    smem_out = jax.ShapeDtypeStruct((1, 1), jnp.float32)
    smem_spec = pl.BlockSpec((1, 1), lambda i: (0, 0), memory_space=pltpu.SMEM)
    zq3, l0, l1, l2, l3 = pl.pallas_call(
        _body,
        grid=(NB // 8,),
        in_specs=[pl.BlockSpec((8, D, T), lambda i: (i, 0, 0)),
                  pl.BlockSpec((K, D), lambda i: (0, 0))],
        out_specs=[pl.BlockSpec((8, D, T), lambda i: (i, 0, 0)),
                   smem_spec, smem_spec, smem_spec, smem_spec],
        out_shape=[jax.ShapeDtypeStruct((NB, D, T), jnp.float32),
                   smem_out, smem_out, smem_out, smem_out],
        scratch_shapes=[pltpu.VMEM((1, K), jnp.float32),
                        pltpu.SMEM((1,), jnp.float32)],
    )(x3, E_weight)

    Zq = zq3.reshape(NB, D, 32, 32)
    return (l0.reshape(()), Zq, l1.reshape(()), l2.reshape(()),
            l3.reshape(()))
